# baseline (speedup 1.0000x reference)
import functools

import jax
import jax.numpy as jnp
from jax import lax
from jax.experimental import pallas as pl
from jax.experimental.pallas import tpu as pltpu
from jax.experimental.pallas import tpu_sc as plsc

NUM_TYPES = 7
DIM = 128
BATCH = 4096
HIST = 200

N = BATCH * HIST
NC = 2
NS = 16
NW = NC * NS
PER_W = N // NW
CHUNK = 400
NCHUNK = PER_W // CHUNK
SPLIT_A = 176
SPLIT_B = CHUNK - SPLIT_A


def _body(idx_hbm, table_hbm, out_hbm, table_s,
          idx_all, table_v, rows0, rows1, gsem, osem0, osem1):
    sid = lax.axis_index("s")
    wid = sid * NC + lax.axis_index("c")
    base = wid * PER_W

    @pl.when(sid == 0)
    def _():
        pltpu.sync_copy(table_hbm, table_s)

    pltpu.sync_copy(idx_hbm.at[pl.ds(base, PER_W)], idx_all)
    pltpu.sync_copy(table_hbm, table_v)
    plsc.subcore_barrier()

    bufs = ((rows0, osem0), (rows1, osem1))
    lane = lax.iota(jnp.int32, 16)

    def gather_start_out(g, b):
        rows_v, osem = bufs[b]
        goff = g * CHUNK
        gdesc = pltpu.make_async_copy(
            table_s.at[idx_all.at[pl.ds(goff, SPLIT_A)]],
            rows_v.at[pl.ds(0, SPLIT_A)],
            gsem,
        )
        gdesc.start()

        def fill_group(m, carry):
            r0 = SPLIT_A + m * 16
            ids16 = idx_all[pl.ds(goff + r0, 16)]
            rowi = r0 + lane
            for c in range(DIM):
                col = jnp.full((16,), c, jnp.int32)
                val = plsc.load_gather(table_v, [ids16, col])
                plsc.store_scatter(rows_v, [rowi, col], val)
            return carry

        lax.fori_loop(0, SPLIT_B // 16, fill_group, 0)
        gdesc.wait()
        pltpu.make_async_copy(
            rows_v, out_hbm.at[pl.ds(base + goff, CHUNK)], osem
        ).start()

    gather_start_out(0, 0)
    gather_start_out(1, 1)

    def pair_step(p, carry):
        for b in range(2):
            g = p * 2 + b
            rows_v, osem = bufs[b]
            pltpu.make_async_copy(
                rows_v, out_hbm.at[pl.ds(base + g * CHUNK, CHUNK)], osem
            ).wait()
            gather_start_out(g, b)
        return carry

    lax.fori_loop(1, NCHUNK // 2, pair_step, 0)

    for b in range(2):
        rows_v, osem = bufs[b]
        off = base + (NCHUNK - 2 + b) * CHUNK
        pltpu.make_async_copy(rows_v, out_hbm.at[pl.ds(off, CHUNK)], osem).wait()


@jax.jit
def _lookup(type_ids_flat, emb_weight):
    mesh = plsc.VectorSubcoreMesh(
        core_axis_name="c", subcore_axis_name="s",
        num_cores=NC, num_subcores=NS,
    )
    return pl.kernel(
        _body,
        out_type=jax.ShapeDtypeStruct((N, DIM), jnp.float32),
        mesh=mesh,
        compiler_params=pltpu.CompilerParams(needs_layout_passes=False),
        scratch_types=[
            pltpu.VMEM_SHARED((NUM_TYPES, DIM), jnp.float32),
            pltpu.VMEM((PER_W,), jnp.int32),
            pltpu.VMEM((NUM_TYPES, DIM), jnp.float32),
            pltpu.VMEM((CHUNK, DIM), jnp.float32),
            pltpu.VMEM((CHUNK, DIM), jnp.float32),
            pltpu.SemaphoreType.DMA,
            pltpu.SemaphoreType.DMA,
            pltpu.SemaphoreType.DMA,
        ],
    )(type_ids_flat, emb_weight)


def kernel(type_ids, emb_weight):
    flat = type_ids.reshape(-1).astype(jnp.int32)
    out = _lookup(flat, emb_weight)
    return out.reshape(BATCH, HIST, DIM)


# speedup vs baseline: 4.2608x; 4.2608x over previous
import functools

import jax
import jax.numpy as jnp
from jax import lax
from jax.experimental import pallas as pl
from jax.experimental.pallas import tpu as pltpu
from jax.experimental.pallas import tpu_sc as plsc

NUM_TYPES = 7
DIM = 128
BATCH = 4096
HIST = 200

N = BATCH * HIST
NC = 2
NS = 16
NW = NC * NS
PER_W = N // NW
CHUNK = 400
NCHUNK = PER_W // CHUNK
SPLIT_A = 176
SPLIT_B = CHUNK - SPLIT_A


def _body(idx_hbm, table_hbm, out_hbm, table_s,
          idx_all, table_v, rows0, rows1, gsem, osem0, osem1):
    sid = lax.axis_index("s")
    wid = sid * NC + lax.axis_index("c")
    base = wid * PER_W

    @pl.when(sid == 0)
    def _():
        pltpu.sync_copy(table_hbm, table_s)

    pltpu.sync_copy(idx_hbm.at[pl.ds(base, PER_W)], idx_all)
    pltpu.sync_copy(table_hbm, table_v)
    plsc.subcore_barrier()

    bufs = ((rows0, osem0), (rows1, osem1))
    lane = lax.iota(jnp.int32, 16)

    def gather_start_out(g, b):
        rows_v, osem = bufs[b]
        goff = g * CHUNK
        gdesc = pltpu.make_async_copy(
            table_s.at[idx_all.at[pl.ds(goff, SPLIT_A)]],
            rows_v.at[pl.ds(0, SPLIT_A)],
            gsem,
        )
        gdesc.start()

        def fill_group(m, carry):
            r0 = SPLIT_A + m * 16
            ids16 = idx_all[pl.ds(goff + r0, 16)]
            rowi = r0 + lane
            def diag_step(d, c2):
                diag = jnp.bitwise_and(lane + d, 15)
                for k in range(DIM // 16):
                    col = diag + (16 * k)
                    val = plsc.load_gather(table_v, [ids16, col])
                    plsc.store_scatter(rows_v, [rowi, col], val)
                return c2

            lax.fori_loop(0, 16, diag_step, 0)
            return carry

        lax.fori_loop(0, SPLIT_B // 16, fill_group, 0)
        gdesc.wait()
        pltpu.make_async_copy(
            rows_v, out_hbm.at[pl.ds(base + goff, CHUNK)], osem
        ).start()

    gather_start_out(0, 0)
    gather_start_out(1, 1)

    def pair_step(p, carry):
        for b in range(2):
            g = p * 2 + b
            rows_v, osem = bufs[b]
            pltpu.make_async_copy(
                rows_v, out_hbm.at[pl.ds(base + g * CHUNK, CHUNK)], osem
            ).wait()
            gather_start_out(g, b)
        return carry

    lax.fori_loop(1, NCHUNK // 2, pair_step, 0)

    for b in range(2):
        rows_v, osem = bufs[b]
        off = base + (NCHUNK - 2 + b) * CHUNK
        pltpu.make_async_copy(rows_v, out_hbm.at[pl.ds(off, CHUNK)], osem).wait()


@jax.jit
def _lookup(type_ids_flat, emb_weight):
    mesh = plsc.VectorSubcoreMesh(
        core_axis_name="c", subcore_axis_name="s",
        num_cores=NC, num_subcores=NS,
    )
    return pl.kernel(
        _body,
        out_type=jax.ShapeDtypeStruct((N, DIM), jnp.float32),
        mesh=mesh,
        compiler_params=pltpu.CompilerParams(needs_layout_passes=False),
        scratch_types=[
            pltpu.VMEM_SHARED((NUM_TYPES, DIM), jnp.float32),
            pltpu.VMEM((PER_W,), jnp.int32),
            pltpu.VMEM((NUM_TYPES, DIM), jnp.float32),
            pltpu.VMEM((CHUNK, DIM), jnp.float32),
            pltpu.VMEM((CHUNK, DIM), jnp.float32),
            pltpu.SemaphoreType.DMA,
            pltpu.SemaphoreType.DMA,
            pltpu.SemaphoreType.DMA,
        ],
    )(type_ids_flat, emb_weight)


def kernel(type_ids, emb_weight):
    flat = type_ids.reshape(-1).astype(jnp.int32)
    out = _lookup(flat, emb_weight)
    return out.reshape(BATCH, HIST, DIM)


# speedup vs baseline: 4.5303x; 1.0633x over previous
import functools

import jax
import jax.numpy as jnp
from jax import lax
from jax.experimental import pallas as pl
from jax.experimental.pallas import tpu as pltpu
from jax.experimental.pallas import tpu_sc as plsc

NUM_TYPES = 7
DIM = 128
BATCH = 4096
HIST = 200

N = BATCH * HIST
NC = 2
NS = 16
NW = NC * NS
PER_W = N // NW
CHUNK = 400
NCHUNK = PER_W // CHUNK
SPLIT_A = 176
SPLIT_B = CHUNK - SPLIT_A


def _body(idx_hbm, table_hbm, tableflat_hbm, out_hbm, table_s,
          idx_all, table_vf, rows0, rows1, gsem, osem0, osem1):
    sid = lax.axis_index("s")
    wid = sid * NC + lax.axis_index("c")
    base = wid * PER_W

    @pl.when(sid == 0)
    def _():
        pltpu.sync_copy(table_hbm, table_s)

    pltpu.sync_copy(idx_hbm.at[pl.ds(base, PER_W)], idx_all)
    pltpu.sync_copy(tableflat_hbm, table_vf)
    plsc.subcore_barrier()

    bufs = ((rows0, osem0), (rows1, osem1))
    lane = lax.iota(jnp.int32, 16)

    def gather_start_out(g, b):
        rows_v, osem = bufs[b]
        goff = g * CHUNK
        gdesc = pltpu.make_async_copy(
            table_s.at[idx_all.at[pl.ds(goff, SPLIT_A)]],
            rows_v.at[pl.ds(0, SPLIT_A)],
            gsem,
        )
        gdesc.start()

        def fill_group(m, carry):
            r0 = SPLIT_A + m * 16
            ids16 = idx_all[pl.ds(goff + r0, 16)]
            idsf = ids16 * DIM
            rowi = r0 + lane
            def diag_step(d4, c2):
                for dd in range(4):
                    diag = jnp.bitwise_and(lane + (d4 * 4 + dd), 15)
                    for k in range(DIM // 16):
                        col = diag + (16 * k)
                        val = plsc.load_gather(table_vf, [idsf + col])
                        plsc.store_scatter(rows_v, [rowi, col], val)
                return c2

            lax.fori_loop(0, 4, diag_step, 0)
            return carry

        lax.fori_loop(0, SPLIT_B // 16, fill_group, 0)
        gdesc.wait()
        pltpu.make_async_copy(
            rows_v, out_hbm.at[pl.ds(base + goff, CHUNK)], osem
        ).start()

    gather_start_out(0, 0)
    gather_start_out(1, 1)

    def pair_step(p, carry):
        for b in range(2):
            g = p * 2 + b
            rows_v, osem = bufs[b]
            pltpu.make_async_copy(
                rows_v, out_hbm.at[pl.ds(base + g * CHUNK, CHUNK)], osem
            ).wait()
            gather_start_out(g, b)
        return carry

    lax.fori_loop(1, NCHUNK // 2, pair_step, 0)

    for b in range(2):
        rows_v, osem = bufs[b]
        off = base + (NCHUNK - 2 + b) * CHUNK
        pltpu.make_async_copy(rows_v, out_hbm.at[pl.ds(off, CHUNK)], osem).wait()


@jax.jit
def _lookup(type_ids_flat, emb_weight):
    mesh = plsc.VectorSubcoreMesh(
        core_axis_name="c", subcore_axis_name="s",
        num_cores=NC, num_subcores=NS,
    )
    return pl.kernel(
        _body,
        out_type=jax.ShapeDtypeStruct((N, DIM), jnp.float32),
        mesh=mesh,
        compiler_params=pltpu.CompilerParams(needs_layout_passes=False),
        scratch_types=[
            pltpu.VMEM_SHARED((NUM_TYPES, DIM), jnp.float32),
            pltpu.VMEM((PER_W,), jnp.int32),
            pltpu.VMEM(((NUM_TYPES + 1) * DIM,), jnp.float32),
            pltpu.VMEM((CHUNK, DIM), jnp.float32),
            pltpu.VMEM((CHUNK, DIM), jnp.float32),
            pltpu.SemaphoreType.DMA,
            pltpu.SemaphoreType.DMA,
            pltpu.SemaphoreType.DMA,
        ],
    )(type_ids_flat, emb_weight,
      jnp.concatenate([emb_weight.reshape(-1), jnp.zeros((DIM,), jnp.float32)]))


def kernel(type_ids, emb_weight):
    flat = type_ids.reshape(-1).astype(jnp.int32)
    out = _lookup(flat, emb_weight)
    return out.reshape(BATCH, HIST, DIM)


# speedup vs baseline: 10.6142x; 2.3429x over previous
import jax
import jax.numpy as jnp
from jax import lax
from jax.experimental import pallas as pl
from jax.experimental.pallas import tpu as pltpu
from jax.experimental.pallas import tpu_sc as plsc

NUM_TYPES = 7
DIM = 128
BATCH = 4096
HIST = 200

N = BATCH * HIST
NC = 2
NS = 16
NW = NC * NS
PER_W = N // NW
CHUNK = 400
NCHUNK = PER_W // CHUNK


def _body(idx_hbm, table_hbm, out_hbm, table_s,
          idx_all, rows0, rows1, gsem, osem0, osem1):
    sid = lax.axis_index("s")
    wid = sid * NC + lax.axis_index("c")
    base = wid * PER_W

    @pl.when(sid == 0)
    def _():
        pltpu.sync_copy(table_hbm, table_s)

    pltpu.sync_copy(idx_hbm.at[pl.ds(base, PER_W)], idx_all)
    plsc.subcore_barrier()

    bufs = ((rows0, osem0), (rows1, osem1))

    def gather_start_out(g, b):
        rows_v, osem = bufs[b]
        pltpu.async_copy(
            table_s.at[idx_all.at[pl.ds(g * CHUNK, CHUNK)]], rows_v, gsem
        ).wait()
        pltpu.make_async_copy(
            rows_v, out_hbm.at[pl.ds(base + g * CHUNK, CHUNK)], osem
        ).start()

    gather_start_out(0, 0)
    gather_start_out(1, 1)

    def pair_step(p, carry):
        for b in range(2):
            g = p * 2 + b
            rows_v, osem = bufs[b]
            pltpu.make_async_copy(
                rows_v, out_hbm.at[pl.ds(base + g * CHUNK, CHUNK)], osem
            ).wait()
            gather_start_out(g, b)
        return carry

    lax.fori_loop(1, NCHUNK // 2, pair_step, 0)

    for b in range(2):
        rows_v, osem = bufs[b]
        off = base + (NCHUNK - 2 + b) * CHUNK
        pltpu.make_async_copy(rows_v, out_hbm.at[pl.ds(off, CHUNK)], osem).wait()


@jax.jit
def _lookup(type_ids_flat, emb_weight):
    mesh = plsc.VectorSubcoreMesh(
        core_axis_name="c", subcore_axis_name="s",
        num_cores=NC, num_subcores=NS,
    )
    return pl.kernel(
        _body,
        out_type=jax.ShapeDtypeStruct((N, DIM), jnp.float32),
        mesh=mesh,
        scratch_types=[
            pltpu.VMEM_SHARED((NUM_TYPES, DIM), jnp.float32),
            pltpu.VMEM((PER_W,), jnp.int32),
            pltpu.VMEM((CHUNK, DIM), jnp.float32),
            pltpu.VMEM((CHUNK, DIM), jnp.float32),
            pltpu.SemaphoreType.DMA,
            pltpu.SemaphoreType.DMA,
            pltpu.SemaphoreType.DMA,
        ],
    )(type_ids_flat, emb_weight)


def kernel(type_ids, emb_weight):
    flat = type_ids.reshape(-1).astype(jnp.int32)
    out = _lookup(flat, emb_weight)
    return out.reshape(BATCH, HIST, DIM)


# speedup vs baseline: 11.1213x; 1.0478x over previous
import jax
import jax.numpy as jnp
from jax import lax
from jax.experimental import pallas as pl
from jax.experimental.pallas import tpu as pltpu
from jax.experimental.pallas import tpu_sc as plsc

NUM_TYPES = 7
DIM = 128
BATCH = 4096
HIST = 200

N = BATCH * HIST
NC = 2
NS = 16
NW = NC * NS
PER_W = N // NW
CHUNK = 200
NCHUNK = PER_W // CHUNK
NSLOT = 4


def _body(idx_hbm, table_hbm, out_hbm, table_s,
          idx_all, rows0, rows1, rows2, rows3,
          gsem0, gsem1, gsem2, gsem3, osem0, osem1, osem2, osem3):
    sid = lax.axis_index("s")
    wid = sid * NC + lax.axis_index("c")
    base = wid * PER_W

    @pl.when(sid == 0)
    def _():
        pltpu.sync_copy(table_hbm, table_s)

    pltpu.sync_copy(idx_hbm.at[pl.ds(base, PER_W)], idx_all)
    plsc.subcore_barrier()

    bufs = ((rows0, gsem0, osem0), (rows1, gsem1, osem1),
            (rows2, gsem2, osem2), (rows3, gsem3, osem3))

    def gdesc(g, b):
        rows_v, gsem, _ = bufs[b]
        return pltpu.make_async_copy(
            table_s.at[idx_all.at[pl.ds(g * CHUNK, CHUNK)]], rows_v, gsem
        )

    def odesc(g, b):
        rows_v, _, osem = bufs[b]
        return pltpu.make_async_copy(
            rows_v, out_hbm.at[pl.ds(base + g * CHUNK, CHUNK)], osem
        )

    def step(g, b, drain, issue):
        gdesc(g, b).wait()
        odesc(g, b).start()
        if drain:
            odesc(g - 1, (b - 1) % NSLOT).wait()
        if issue:
            gdesc(g + NSLOT - 1, (b - 1) % NSLOT).start()

    for g in range(NSLOT - 1):
        gdesc(g, g).start()
    for g in range(NSLOT):
        step(g, g % NSLOT, drain=g >= 1, issue=True)

    def quad_step(p, carry):
        for b in range(NSLOT):
            g = p * NSLOT + b
            step(g, b, drain=True, issue=True)
        return carry

    _tail = NSLOT * ((NCHUNK - NSLOT + 1) // NSLOT)
    lax.fori_loop(1, (NCHUNK - NSLOT + 1) // NSLOT, quad_step, 0)

    for g in range(_tail, NCHUNK):
        step(g, g % NSLOT, drain=True, issue=g + NSLOT - 1 < NCHUNK)
    odesc(NCHUNK - 1, (NCHUNK - 1) % NSLOT).wait()


@jax.jit
def _lookup(type_ids_flat, emb_weight):
    mesh = plsc.VectorSubcoreMesh(
        core_axis_name="c", subcore_axis_name="s",
        num_cores=NC, num_subcores=NS,
    )
    return pl.kernel(
        _body,
        out_type=jax.ShapeDtypeStruct((N, DIM), jnp.float32),
        mesh=mesh,
        scratch_types=[
            pltpu.VMEM_SHARED((NUM_TYPES, DIM), jnp.float32),
            pltpu.VMEM((PER_W,), jnp.int32),
        ] + [pltpu.VMEM((CHUNK, DIM), jnp.float32)] * NSLOT
          + [pltpu.SemaphoreType.DMA] * (2 * NSLOT),
    )(type_ids_flat, emb_weight)


def kernel(type_ids, emb_weight):
    flat = type_ids.reshape(-1).astype(jnp.int32)
    out = _lookup(flat, emb_weight)
    return out.reshape(BATCH, HIST, DIM)
